# baseline (speedup 1.0000x reference)
import functools

import jax
import jax.numpy as jnp
from jax import lax
from jax.experimental import pallas as pl
from jax.experimental.pallas import tpu as pltpu
from jax.experimental.pallas import tpu_sc as plsc

K = 10
KPAD = 10
_NEG_INF = float("-inf")

_SC_CORES = 2
_SC_SUBCORES = 16
_SC_WORKERS = _SC_CORES * _SC_SUBCORES
_GCHUNK = 128


def _topk_body(x_ref, idx_ref, *, nblk):
    x = x_ref[0]
    C, N = x.shape
    bs = N // nblk
    base = pl.program_id(0) * N
    sq = jnp.sum(x * x, axis=0, keepdims=True)
    x_bf = x.astype(jnp.bfloat16)
    for r in range(nblk):
        xb = x[:, r * bs:(r + 1) * bs]
        g = lax.dot_general(xb.astype(jnp.bfloat16), x_bf,
                            (((0,), (0,)), ((), ())),
                            preferred_element_type=jnp.float32)
        d = 2.0 * g - sq
        iot = lax.broadcasted_iota(jnp.int32, (bs, N), 1)
        cs = []
        for _ in range(K):
            mx = jnp.max(d, axis=1, keepdims=True)
            eq = d == mx
            c = jnp.min(jnp.where(eq, iot, N), axis=1, keepdims=True)
            cs.append(c)
            d = jnp.where(iot == c, _NEG_INF, d)
        idxmat = jnp.concatenate(cs, axis=1)
        idx_ref[0, r * bs:(r + 1) * bs, :] = idxmat + base


def _topk(x, *, nblk=2):
    B, C, N = x.shape
    return pl.pallas_call(
        functools.partial(_topk_body, nblk=nblk),
        grid=(B,),
        in_specs=[pl.BlockSpec((1, C, N), lambda b: (b, 0, 0))],
        out_specs=pl.BlockSpec((1, N, KPAD), lambda b: (b, 0, 0)),
        out_shape=jax.ShapeDtypeStruct((B, N, KPAD), jnp.int32),
    )(x)


def _sc_gather(table, idx):
    R = idx.shape[0]
    C = table.shape[1]
    per_w = R // _SC_WORKERS
    nchunk = per_w // _GCHUNK
    mesh = plsc.VectorSubcoreMesh(core_axis_name="c", subcore_axis_name="s")

    @functools.partial(
        pl.kernel, mesh=mesh,
        out_type=jax.ShapeDtypeStruct((R, C), jnp.float32),
        scratch_types=[
            pltpu.VMEM((nchunk, _GCHUNK), jnp.int32),
            pltpu.VMEM((nchunk, _GCHUNK, C), jnp.float32),
            pltpu.SemaphoreType.DMA,
            pltpu.SemaphoreType.DMA,
        ],
    )
    def gather_k(table_hbm, idx_hbm, out_hbm, ivs, rvs, gsem, wsem):
        wid = lax.axis_index("s") * _SC_CORES + lax.axis_index("c")
        wbase = wid * per_w
        ghs = []
        for ch in range(nchunk):
            iv = ivs.at[ch]
            pltpu.sync_copy(idx_hbm.at[pl.ds(wbase + ch * _GCHUNK, _GCHUNK)],
                            iv)
            ghs.append(pltpu.async_copy(table_hbm.at[iv], rvs.at[ch], gsem))
        for gh in ghs:
            gh.wait()
        whs = []
        for ch in range(nchunk):
            whs.append(pltpu.async_copy(
                rvs.at[ch],
                out_hbm.at[pl.ds(wbase + ch * _GCHUNK, _GCHUNK)], wsem))
        for wh in whs:
            wh.wait()

    return gather_k(table, idx)


def _conv_body(feat_ref, xt_ref, w_ref, pre_ref):
    fr = feat_ref[0]
    xr = xt_ref[0]
    C = xr.shape[1]
    w_bf = w_ref[...].astype(jnp.bfloat16)
    xr_bf = xr.astype(jnp.bfloat16)
    acc = jnp.zeros((w_bf.shape[0], xr.shape[0]), jnp.float32)
    for t in range(K):
        nbr = fr[:, t * C:(t + 1) * C]
        ff = jnp.concatenate([(nbr - xr).astype(jnp.bfloat16), xr_bf],
                             axis=1)
        acc = acc + lax.dot_general(w_bf, ff, (((1,), (1,)), ((), ())),
                                    preferred_element_type=jnp.float32)
    pre_ref[0] = acc / float(K)


def _conv(feat, xt, w, *, nblk=8):
    B, N, _ = feat.shape
    C = xt.shape[2]
    O = w.shape[0]
    bs = N // nblk
    return pl.pallas_call(
        _conv_body,
        grid=(B, nblk),
        in_specs=[
            pl.BlockSpec((1, bs, KPAD * C), lambda b, r: (b, r, 0)),
            pl.BlockSpec((1, bs, C), lambda b, r: (b, r, 0)),
            pl.BlockSpec((O, 2 * C), lambda b, r: (0, 0)),
        ],
        out_specs=pl.BlockSpec((1, O, bs), lambda b, r: (b, 0, r)),
        out_shape=jax.ShapeDtypeStruct((B, O, N), jnp.float32),
    )(feat, xt, w)


def _gcn_body(pre_ref, res_ref, out_ref):
    p = pre_ref[0]
    N = p.shape[1]
    mu = jnp.mean(p, axis=1, keepdims=True)
    dev = p - mu
    var = jnp.sum(dev * dev, axis=1, keepdims=True) / (N - 1)
    y = dev / jnp.sqrt(var + 0.001)
    y = jnp.maximum(y, 0.0)
    if res_ref is not None:
        y = y + res_ref[0]
    out_ref[0] = y


def _gcn(pre, residual):
    B, O, N = pre.shape
    if residual is None:
        def kern(pre_ref, out_ref):
            _gcn_body(pre_ref, None, out_ref)
        operands = (pre,)
        in_specs = [pl.BlockSpec((1, O, N), lambda b: (b, 0, 0))]
    else:
        kern = _gcn_body
        operands = (pre, residual)
        in_specs = [pl.BlockSpec((1, O, N), lambda b: (b, 0, 0)),
                    pl.BlockSpec((1, O, N), lambda b: (b, 0, 0))]
    return pl.pallas_call(
        kern,
        grid=(B,),
        in_specs=in_specs,
        out_specs=pl.BlockSpec((1, O, N), lambda b: (b, 0, 0)),
        out_shape=jax.ShapeDtypeStruct((B, O, N), jnp.float32),
    )(*operands)


def _layer_b(xb, xtb, w, residual_b):
    _, C, N = xb.shape
    idx = _topk(xb)
    feat = _sc_gather(xtb.reshape(N, C), idx.reshape(N * KPAD))
    pre = _conv(feat.reshape(1, N, KPAD * C), xtb, w)
    return _gcn(pre, residual_b)


def kernel(x, W1, b1, W2, b2):
    del b1, b2
    B = x.shape[0]
    outs = []
    for b in range(B):
        xb = x[b:b + 1]
        hb = _layer_b(xb, jnp.transpose(xb, (0, 2, 1)), W1, None)
        outs.append(_layer_b(hb, jnp.transpose(hb, (0, 2, 1)), W2, xb))
    return jnp.concatenate(outs, axis=0)


# speedup vs baseline: 1.0736x; 1.0736x over previous
import functools

import jax
import jax.numpy as jnp
from jax import lax
from jax.experimental import pallas as pl
from jax.experimental.pallas import tpu as pltpu
from jax.experimental.pallas import tpu_sc as plsc

K = 10
KPAD = 10
_NEG_INF = float("-inf")

_SC_CORES = 2
_SC_SUBCORES = 16
_SC_WORKERS = _SC_CORES * _SC_SUBCORES
_GCHUNK = 128


def _topk_body(x_ref, idx_ref, *, nblk):
    x = x_ref[0]
    C, N = x.shape
    bs = N // nblk
    base = pl.program_id(0) * N
    sq = jnp.sum(x * x, axis=0, keepdims=True)
    x_bf = x.astype(jnp.bfloat16)
    for r in range(nblk):
        xb = x[:, r * bs:(r + 1) * bs]
        g = lax.dot_general(xb.astype(jnp.bfloat16), x_bf,
                            (((0,), (0,)), ((), ())),
                            preferred_element_type=jnp.float32)
        d = 2.0 * g - sq
        iot = lax.broadcasted_iota(jnp.int32, (bs, N), 1)
        cs = []
        for _ in range(K):
            mx = jnp.max(d, axis=1, keepdims=True)
            eq = d == mx
            c = jnp.min(jnp.where(eq, iot, N), axis=1, keepdims=True)
            cs.append(c)
            d = jnp.where(iot == c, _NEG_INF, d)
        idxmat = jnp.concatenate(cs, axis=1)
        idx_ref[0, r * bs:(r + 1) * bs, :] = idxmat + base


def _topk(x, *, nblk=2):
    B, C, N = x.shape
    return pl.pallas_call(
        functools.partial(_topk_body, nblk=nblk),
        grid=(B,),
        in_specs=[pl.BlockSpec((1, C, N), lambda b: (b, 0, 0))],
        out_specs=pl.BlockSpec((1, N, KPAD), lambda b: (b, 0, 0)),
        out_shape=jax.ShapeDtypeStruct((B, N, KPAD), jnp.int32),
    )(x)


def _sc_gather(table, idx):
    R = idx.shape[0]
    C = table.shape[1]
    per_w = R // _SC_WORKERS
    nchunk = per_w // _GCHUNK
    mesh = plsc.VectorSubcoreMesh(core_axis_name="c", subcore_axis_name="s")

    @functools.partial(
        pl.kernel, mesh=mesh,
        out_type=jax.ShapeDtypeStruct((R, C), jnp.float32),
        scratch_types=[
            pltpu.VMEM((nchunk, _GCHUNK), jnp.int32),
            pltpu.VMEM((nchunk, _GCHUNK, C), jnp.float32),
            pltpu.SemaphoreType.DMA,
            pltpu.SemaphoreType.DMA,
        ],
    )
    def gather_k(table_hbm, idx_hbm, out_hbm, ivs, rvs, gsem, wsem):
        wid = lax.axis_index("s") * _SC_CORES + lax.axis_index("c")
        wbase = wid * per_w
        ghs = []
        for ch in range(nchunk):
            iv = ivs.at[ch]
            pltpu.sync_copy(idx_hbm.at[pl.ds(wbase + ch * _GCHUNK, _GCHUNK)],
                            iv)
            ghs.append(pltpu.async_copy(table_hbm.at[iv], rvs.at[ch], gsem))
        for gh in ghs:
            gh.wait()
        whs = []
        for ch in range(nchunk):
            whs.append(pltpu.async_copy(
                rvs.at[ch],
                out_hbm.at[pl.ds(wbase + ch * _GCHUNK, _GCHUNK)], wsem))
        for wh in whs:
            wh.wait()

    return gather_k(table, idx)


def _conv_body(feat_ref, xt_ref, w_ref, pre_ref):
    fr = feat_ref[0]
    xr = xt_ref[0]
    C = xr.shape[1]
    w_bf = w_ref[...].astype(jnp.bfloat16)
    xr_bf = xr.astype(jnp.bfloat16)
    acc = jnp.zeros((w_bf.shape[0], xr.shape[0]), jnp.float32)
    for t in range(K):
        nbr = fr[:, t * C:(t + 1) * C]
        ff = jnp.concatenate([(nbr - xr).astype(jnp.bfloat16), xr_bf],
                             axis=1)
        acc = acc + lax.dot_general(w_bf, ff, (((1,), (1,)), ((), ())),
                                    preferred_element_type=jnp.float32)
    pre_ref[0] = acc / float(K)


def _conv(feat, xt, w, *, nblk=8):
    B, N, _ = feat.shape
    C = xt.shape[2]
    O = w.shape[0]
    bs = N // nblk
    return pl.pallas_call(
        _conv_body,
        grid=(B, nblk),
        in_specs=[
            pl.BlockSpec((1, bs, KPAD * C), lambda b, r: (b, r, 0)),
            pl.BlockSpec((1, bs, C), lambda b, r: (b, r, 0)),
            pl.BlockSpec((O, 2 * C), lambda b, r: (0, 0)),
        ],
        out_specs=pl.BlockSpec((1, O, bs), lambda b, r: (b, 0, r)),
        out_shape=jax.ShapeDtypeStruct((B, O, N), jnp.float32),
    )(feat, xt, w)


def _gcn_body(pre_ref, res_ref, out_ref):
    p = pre_ref[0]
    N = p.shape[1]
    mu = jnp.mean(p, axis=1, keepdims=True)
    dev = p - mu
    var = jnp.sum(dev * dev, axis=1, keepdims=True) / (N - 1)
    y = dev / jnp.sqrt(var + 0.001)
    y = jnp.maximum(y, 0.0)
    if res_ref is not None:
        y = y + res_ref[0]
    out_ref[0] = y


def _gcn(pre, residual):
    B, O, N = pre.shape
    if residual is None:
        def kern(pre_ref, out_ref):
            _gcn_body(pre_ref, None, out_ref)
        operands = (pre,)
        in_specs = [pl.BlockSpec((1, O, N), lambda b: (b, 0, 0))]
    else:
        kern = _gcn_body
        operands = (pre, residual)
        in_specs = [pl.BlockSpec((1, O, N), lambda b: (b, 0, 0)),
                    pl.BlockSpec((1, O, N), lambda b: (b, 0, 0))]
    return pl.pallas_call(
        kern,
        grid=(B,),
        in_specs=in_specs,
        out_specs=pl.BlockSpec((1, O, N), lambda b: (b, 0, 0)),
        out_shape=jax.ShapeDtypeStruct((B, O, N), jnp.float32),
    )(*operands)


def _layer(x, xt, w, residual):
    B, C, N = x.shape
    table = xt.reshape(B * N, C)
    idxs = [_topk(x[b:b + 1]) for b in range(B)]
    feats = [_sc_gather(table, idxs[b].reshape(N * KPAD) + b * N)
             for b in range(B)]
    outs = []
    for b in range(B):
        pre = _conv(feats[b].reshape(1, N, KPAD * C), xt[b:b + 1], w)
        outs.append(_gcn(pre, None if residual is None
                         else residual[b:b + 1]))
    return jnp.concatenate(outs, axis=0)


def kernel(x, W1, b1, W2, b2):
    del b1, b2
    xt = jnp.transpose(x, (0, 2, 1))
    h = _layer(x, xt, W1, None)
    ht = jnp.transpose(h, (0, 2, 1))
    return _layer(h, ht, W2, x)


# speedup vs baseline: 1.1120x; 1.0357x over previous
import functools

import jax
import jax.numpy as jnp
from jax import lax
from jax.experimental import pallas as pl
from jax.experimental.pallas import tpu as pltpu
from jax.experimental.pallas import tpu_sc as plsc

K = 10
KPAD = 10
_NEG_INF = float("-inf")

_SC_CORES = 2
_SC_SUBCORES = 16
_SC_WORKERS = _SC_CORES * _SC_SUBCORES
_GCHUNK = 128


def _topk_body(x_ref, idx_ref, *, nblk):
    x = x_ref[0]
    C, N = x.shape
    bs = N // nblk
    base = pl.program_id(0) * N
    sq = jnp.sum(x * x, axis=0, keepdims=True)
    x_bf = x.astype(jnp.bfloat16)
    ji = lax.broadcasted_iota(jnp.int32, (N, 2), 0)
    iota2 = jnp.concatenate(
        [ji[:, 0:1] // 128, ji[:, 1:2] % 128],
        axis=1).astype(jnp.float32).astype(jnp.bfloat16)
    for r in range(nblk):
        xb = x[:, r * bs:(r + 1) * bs]
        g = lax.dot_general(xb.astype(jnp.bfloat16), x_bf,
                            (((0,), (0,)), ((), ())),
                            preferred_element_type=jnp.float32)
        d = 2.0 * g - sq
        cs = []
        for _ in range(K):
            mx = jnp.max(d, axis=1, keepdims=True)
            eq = d == mx
            eqf = jnp.where(eq, 1.0, 0.0)
            hl = lax.dot_general(eqf.astype(jnp.bfloat16), iota2,
                                 (((1,), (0,)), ((), ())),
                                 preferred_element_type=jnp.float32)
            c = hl[:, 0:1] * 128.0 + hl[:, 1:2]
            cs.append(c)
            d = jnp.where(eq, _NEG_INF, d)
        idxmat = jnp.concatenate(cs, axis=1)
        idxmat = jnp.minimum(idxmat, float(N - 1)).astype(jnp.int32)
        idx_ref[0, r * bs:(r + 1) * bs, :] = idxmat + base


def _topk(x, *, nblk=4):
    B, C, N = x.shape
    return pl.pallas_call(
        functools.partial(_topk_body, nblk=nblk),
        grid=(B,),
        in_specs=[pl.BlockSpec((1, C, N), lambda b: (b, 0, 0))],
        out_specs=pl.BlockSpec((1, N, KPAD), lambda b: (b, 0, 0)),
        out_shape=jax.ShapeDtypeStruct((B, N, KPAD), jnp.int32),
    )(x)


def _sc_gather(table, idx):
    R = idx.shape[0]
    C = table.shape[1]
    per_w = R // _SC_WORKERS
    nchunk = per_w // _GCHUNK
    mesh = plsc.VectorSubcoreMesh(core_axis_name="c", subcore_axis_name="s")

    @functools.partial(
        pl.kernel, mesh=mesh,
        out_type=jax.ShapeDtypeStruct((R, C), jnp.float32),
        scratch_types=[
            pltpu.VMEM((nchunk, _GCHUNK), jnp.int32),
            pltpu.VMEM((nchunk, _GCHUNK, C), jnp.float32),
            pltpu.SemaphoreType.DMA,
            pltpu.SemaphoreType.DMA,
        ],
    )
    def gather_k(table_hbm, idx_hbm, out_hbm, ivs, rvs, gsem, wsem):
        wid = lax.axis_index("s") * _SC_CORES + lax.axis_index("c")
        wbase = wid * per_w
        ghs = []
        for ch in range(nchunk):
            iv = ivs.at[ch]
            pltpu.sync_copy(idx_hbm.at[pl.ds(wbase + ch * _GCHUNK, _GCHUNK)],
                            iv)
            ghs.append(pltpu.async_copy(table_hbm.at[iv], rvs.at[ch], gsem))
        for gh in ghs:
            gh.wait()
        whs = []
        for ch in range(nchunk):
            whs.append(pltpu.async_copy(
                rvs.at[ch],
                out_hbm.at[pl.ds(wbase + ch * _GCHUNK, _GCHUNK)], wsem))
        for wh in whs:
            wh.wait()

    return gather_k(table, idx)


def _conv_body(feat_ref, xt_ref, w_ref, pre_ref):
    fr = feat_ref[0]
    xr = xt_ref[0]
    C = xr.shape[1]
    w_bf = w_ref[...].astype(jnp.bfloat16)
    xr_bf = xr.astype(jnp.bfloat16)
    acc = jnp.zeros((w_bf.shape[0], xr.shape[0]), jnp.float32)
    for t in range(K):
        nbr = fr[:, t * C:(t + 1) * C]
        ff = jnp.concatenate([(nbr - xr).astype(jnp.bfloat16), xr_bf],
                             axis=1)
        acc = acc + lax.dot_general(w_bf, ff, (((1,), (1,)), ((), ())),
                                    preferred_element_type=jnp.float32)
    pre_ref[0] = acc / float(K)


def _conv(feat, xt, w, *, nblk=8):
    B, N, _ = feat.shape
    C = xt.shape[2]
    O = w.shape[0]
    bs = N // nblk
    return pl.pallas_call(
        _conv_body,
        grid=(B, nblk),
        in_specs=[
            pl.BlockSpec((1, bs, KPAD * C), lambda b, r: (b, r, 0)),
            pl.BlockSpec((1, bs, C), lambda b, r: (b, r, 0)),
            pl.BlockSpec((O, 2 * C), lambda b, r: (0, 0)),
        ],
        out_specs=pl.BlockSpec((1, O, bs), lambda b, r: (b, 0, r)),
        out_shape=jax.ShapeDtypeStruct((B, O, N), jnp.float32),
    )(feat, xt, w)


def _gcn_body(pre_ref, res_ref, out_ref):
    p = pre_ref[0]
    N = p.shape[1]
    mu = jnp.mean(p, axis=1, keepdims=True)
    dev = p - mu
    var = jnp.sum(dev * dev, axis=1, keepdims=True) / (N - 1)
    y = dev / jnp.sqrt(var + 0.001)
    y = jnp.maximum(y, 0.0)
    if res_ref is not None:
        y = y + res_ref[0]
    out_ref[0] = y


def _gcn(pre, residual):
    B, O, N = pre.shape
    if residual is None:
        def kern(pre_ref, out_ref):
            _gcn_body(pre_ref, None, out_ref)
        operands = (pre,)
        in_specs = [pl.BlockSpec((1, O, N), lambda b: (b, 0, 0))]
    else:
        kern = _gcn_body
        operands = (pre, residual)
        in_specs = [pl.BlockSpec((1, O, N), lambda b: (b, 0, 0)),
                    pl.BlockSpec((1, O, N), lambda b: (b, 0, 0))]
    return pl.pallas_call(
        kern,
        grid=(B,),
        in_specs=in_specs,
        out_specs=pl.BlockSpec((1, O, N), lambda b: (b, 0, 0)),
        out_shape=jax.ShapeDtypeStruct((B, O, N), jnp.float32),
    )(*operands)


def _layer(x, xt, w, residual):
    B, C, N = x.shape
    table = xt.reshape(B * N, C)
    idxs = [_topk(x[b:b + 1]) for b in range(B)]
    feats = [_sc_gather(table, idxs[b].reshape(N * KPAD) + b * N)
             for b in range(B)]
    outs = []
    for b in range(B):
        pre = _conv(feats[b].reshape(1, N, KPAD * C), xt[b:b + 1], w)
        outs.append(_gcn(pre, None if residual is None
                         else residual[b:b + 1]))
    return jnp.concatenate(outs, axis=0)


def kernel(x, W1, b1, W2, b2):
    del b1, b2
    xt = jnp.transpose(x, (0, 2, 1))
    h = _layer(x, xt, W1, None)
    ht = jnp.transpose(h, (0, 2, 1))
    return _layer(h, ht, W2, x)


# speedup vs baseline: 1.1366x; 1.0221x over previous
import functools

import jax
import jax.numpy as jnp
from jax import lax
from jax.experimental import pallas as pl
from jax.experimental.pallas import tpu as pltpu
from jax.experimental.pallas import tpu_sc as plsc

K = 10
KPAD = 10
_NEG_INF = float("-inf")

_SC_CORES = 2
_SC_SUBCORES = 16
_SC_WORKERS = _SC_CORES * _SC_SUBCORES
_GCHUNK = 128


def _topk_body(x_ref, idx_ref, *, nblk):
    x = x_ref[0]
    C, N = x.shape
    bs = N // nblk
    base = pl.program_id(0) * N
    sq = jnp.sum(x * x, axis=0, keepdims=True)
    x_bf = x.astype(jnp.bfloat16)
    ji = lax.broadcasted_iota(jnp.int32, (N, 2), 0)
    iota2 = jnp.concatenate(
        [ji[:, 0:1] // 128, ji[:, 1:2] % 128],
        axis=1).astype(jnp.float32).astype(jnp.bfloat16)
    for r in range(nblk):
        xb = x[:, r * bs:(r + 1) * bs]
        g = lax.dot_general(xb.astype(jnp.bfloat16), x_bf,
                            (((0,), (0,)), ((), ())),
                            preferred_element_type=jnp.float32)
        d = 2.0 * g - sq
        cs = []
        for _ in range(K):
            mx = jnp.max(d, axis=1, keepdims=True)
            eq = d == mx
            eqf = jnp.where(eq, 1.0, 0.0)
            hl = lax.dot_general(eqf.astype(jnp.bfloat16), iota2,
                                 (((1,), (0,)), ((), ())),
                                 preferred_element_type=jnp.float32)
            c = hl[:, 0:1] * 128.0 + hl[:, 1:2]
            cs.append(c)
            d = jnp.where(eq, _NEG_INF, d)
        idxmat = jnp.concatenate(cs, axis=1)
        idxmat = jnp.minimum(idxmat, float(N - 1)).astype(jnp.int32)
        idx_ref[0, r * bs:(r + 1) * bs, :] = idxmat + base


def _topk(x, *, nblk=4):
    B, C, N = x.shape
    return pl.pallas_call(
        functools.partial(_topk_body, nblk=nblk),
        grid=(B,),
        in_specs=[pl.BlockSpec((1, C, N), lambda b: (b, 0, 0))],
        out_specs=pl.BlockSpec((1, N, KPAD), lambda b: (b, 0, 0)),
        out_shape=jax.ShapeDtypeStruct((B, N, KPAD), jnp.int32),
    )(x)


def _sc_gather(table, idx):
    R = idx.shape[0]
    C = table.shape[1]
    per_w = R // _SC_WORKERS
    nchunk = per_w // _GCHUNK
    mesh = plsc.VectorSubcoreMesh(core_axis_name="c", subcore_axis_name="s")

    @functools.partial(
        pl.kernel, mesh=mesh,
        out_type=jax.ShapeDtypeStruct((R, C), jnp.float32),
        scratch_types=[
            pltpu.VMEM((nchunk, _GCHUNK), jnp.int32),
            pltpu.VMEM((nchunk, _GCHUNK, C), jnp.float32),
            pltpu.SemaphoreType.DMA,
            pltpu.SemaphoreType.DMA,
        ],
    )
    def gather_k(table_hbm, idx_hbm, out_hbm, ivs, rvs, gsem, wsem):
        wid = lax.axis_index("s") * _SC_CORES + lax.axis_index("c")
        wbase = wid * per_w
        ghs = []
        for ch in range(nchunk):
            iv = ivs.at[ch]
            pltpu.sync_copy(idx_hbm.at[pl.ds(wbase + ch * _GCHUNK, _GCHUNK)],
                            iv)
            ghs.append(pltpu.async_copy(table_hbm.at[iv], rvs.at[ch], gsem))
        for gh in ghs:
            gh.wait()
        whs = []
        for ch in range(nchunk):
            whs.append(pltpu.async_copy(
                rvs.at[ch],
                out_hbm.at[pl.ds(wbase + ch * _GCHUNK, _GCHUNK)], wsem))
        for wh in whs:
            wh.wait()

    return gather_k(table, idx)


def _conv_body(feat_ref, xt_ref, w_ref, pre_ref):
    fr = feat_ref[0]
    xr = xt_ref[0]
    C = xr.shape[1]
    w_bf = w_ref[...].astype(jnp.bfloat16)
    xr_bf = xr.astype(jnp.bfloat16)
    acc = jnp.zeros((w_bf.shape[0], xr.shape[0]), jnp.float32)
    for t in range(K):
        nbr = fr[:, t * C:(t + 1) * C]
        ff = jnp.concatenate([(nbr - xr).astype(jnp.bfloat16), xr_bf],
                             axis=1)
        acc = acc + lax.dot_general(w_bf, ff, (((1,), (1,)), ((), ())),
                                    preferred_element_type=jnp.float32)
    pre_ref[0] = acc / float(K)


def _conv(feat, xt, w, *, nblk=4):
    B, N, _ = feat.shape
    C = xt.shape[2]
    O = w.shape[0]
    bs = N // nblk
    return pl.pallas_call(
        _conv_body,
        grid=(B, nblk),
        in_specs=[
            pl.BlockSpec((1, bs, KPAD * C), lambda b, r: (b, r, 0)),
            pl.BlockSpec((1, bs, C), lambda b, r: (b, r, 0)),
            pl.BlockSpec((O, 2 * C), lambda b, r: (0, 0)),
        ],
        out_specs=pl.BlockSpec((1, O, bs), lambda b, r: (b, 0, r)),
        out_shape=jax.ShapeDtypeStruct((B, O, N), jnp.float32),
    )(feat, xt, w)


def _gcn_body(pre_ref, res_ref, out_ref):
    p = pre_ref[0]
    N = p.shape[1]
    mu = jnp.mean(p, axis=1, keepdims=True)
    dev = p - mu
    var = jnp.sum(dev * dev, axis=1, keepdims=True) / (N - 1)
    y = dev / jnp.sqrt(var + 0.001)
    y = jnp.maximum(y, 0.0)
    if res_ref is not None:
        y = y + res_ref[0]
    out_ref[0] = y


def _gcn(pre, residual):
    B, O, N = pre.shape
    if residual is None:
        def kern(pre_ref, out_ref):
            _gcn_body(pre_ref, None, out_ref)
        operands = (pre,)
        in_specs = [pl.BlockSpec((1, O, N), lambda b: (b, 0, 0))]
    else:
        kern = _gcn_body
        operands = (pre, residual)
        in_specs = [pl.BlockSpec((1, O, N), lambda b: (b, 0, 0)),
                    pl.BlockSpec((1, O, N), lambda b: (b, 0, 0))]
    return pl.pallas_call(
        kern,
        grid=(B,),
        in_specs=in_specs,
        out_specs=pl.BlockSpec((1, O, N), lambda b: (b, 0, 0)),
        out_shape=jax.ShapeDtypeStruct((B, O, N), jnp.float32),
    )(*operands)


def _layer(x, xt, w, residual):
    B, C, N = x.shape
    table = xt.reshape(B * N, C)
    idxs = [_topk(x[b:b + 1]) for b in range(B)]
    feats = [_sc_gather(table, idxs[b].reshape(N * KPAD) + b * N)
             for b in range(B)]
    outs = []
    for b in range(B):
        pre = _conv(feats[b].reshape(1, N, KPAD * C), xt[b:b + 1], w)
        outs.append(_gcn(pre, None if residual is None
                         else residual[b:b + 1]))
    return jnp.concatenate(outs, axis=0)


def kernel(x, W1, b1, W2, b2):
    del b1, b2
    xt = jnp.transpose(x, (0, 2, 1))
    h = _layer(x, xt, W1, None)
    ht = jnp.transpose(h, (0, 2, 1))
    return _layer(h, ht, W2, x)


# speedup vs baseline: 1.1423x; 1.0051x over previous
import functools

import jax
import jax.numpy as jnp
from jax import lax
from jax.experimental import pallas as pl
from jax.experimental.pallas import tpu as pltpu
from jax.experimental.pallas import tpu_sc as plsc

K = 10
KPAD = 10
_NEG_INF = float("-inf")

_SC_CORES = 2
_SC_SUBCORES = 16
_SC_WORKERS = _SC_CORES * _SC_SUBCORES
_GCHUNK = 128


def _topk_body(x_ref, idx_ref, *, nblk):
    x = x_ref[0]
    C, N = x.shape
    bs = N // nblk
    base = pl.program_id(0) * N
    sq = jnp.sum(x * x, axis=0, keepdims=True)
    x_bf = x.astype(jnp.bfloat16)
    ji = lax.broadcasted_iota(jnp.int32, (N, 2), 0)
    iota2 = jnp.concatenate(
        [ji[:, 0:1] // 128, ji[:, 1:2] % 128],
        axis=1).astype(jnp.float32).astype(jnp.bfloat16)
    for r in range(nblk):
        xb = x[:, r * bs:(r + 1) * bs]
        g = lax.dot_general(xb.astype(jnp.bfloat16), x_bf,
                            (((0,), (0,)), ((), ())),
                            preferred_element_type=jnp.float32)
        d = 2.0 * g - sq
        cs = []
        for _ in range(K):
            mx = jnp.max(d, axis=1, keepdims=True)
            eq = d == mx
            eqf = jnp.where(eq, 1.0, 0.0)
            hl = lax.dot_general(eqf.astype(jnp.bfloat16), iota2,
                                 (((1,), (0,)), ((), ())),
                                 preferred_element_type=jnp.float32)
            c = hl[:, 0:1] * 128.0 + hl[:, 1:2]
            cs.append(c)
            d = jnp.where(eq, _NEG_INF, d)
        idxmat = jnp.concatenate(cs, axis=1)
        idxmat = jnp.minimum(idxmat, float(N - 1)).astype(jnp.int32)
        idx_ref[0, r * bs:(r + 1) * bs, :] = idxmat + base


def _topk(x, *, nblk=4):
    B, C, N = x.shape
    return pl.pallas_call(
        functools.partial(_topk_body, nblk=nblk),
        grid=(B,),
        in_specs=[pl.BlockSpec((1, C, N), lambda b: (b, 0, 0))],
        out_specs=pl.BlockSpec((1, N, KPAD), lambda b: (b, 0, 0)),
        out_shape=jax.ShapeDtypeStruct((B, N, KPAD), jnp.int32),
    )(x)


def _sc_gather(table, idx):
    R = idx.shape[0]
    C = table.shape[1]
    per_w = R // _SC_WORKERS
    nchunk = per_w // _GCHUNK
    mesh = plsc.VectorSubcoreMesh(core_axis_name="c", subcore_axis_name="s")

    @functools.partial(
        pl.kernel, mesh=mesh,
        out_type=jax.ShapeDtypeStruct((R, C), jnp.float32),
        scratch_types=[
            pltpu.VMEM((nchunk, _GCHUNK), jnp.int32),
            pltpu.VMEM((nchunk, _GCHUNK, C), jnp.float32),
            pltpu.SemaphoreType.DMA,
            pltpu.SemaphoreType.DMA,
        ],
    )
    def gather_k(table_hbm, idx_hbm, out_hbm, ivs, rvs, gsem, wsem):
        wid = lax.axis_index("s") * _SC_CORES + lax.axis_index("c")
        wbase = wid * per_w
        ghs = []
        for ch in range(nchunk):
            iv = ivs.at[ch]
            pltpu.sync_copy(idx_hbm.at[pl.ds(wbase + ch * _GCHUNK, _GCHUNK)],
                            iv)
            ghs.append(pltpu.async_copy(table_hbm.at[iv], rvs.at[ch], gsem))
        for gh in ghs:
            gh.wait()
        whs = []
        for ch in range(nchunk):
            whs.append(pltpu.async_copy(
                rvs.at[ch],
                out_hbm.at[pl.ds(wbase + ch * _GCHUNK, _GCHUNK)], wsem))
        for wh in whs:
            wh.wait()

    return gather_k(table, idx)


def _conv_body(feat_ref, xt_ref, w_ref, pre_ref):
    fr = feat_ref[0]
    xr = xt_ref[0]
    C = xr.shape[1]
    w_bf = w_ref[...].astype(jnp.bfloat16)
    xr_bf = xr.astype(jnp.bfloat16)
    acc = jnp.zeros((w_bf.shape[0], xr.shape[0]), jnp.float32)
    for t in range(K):
        nbr = fr[:, t * C:(t + 1) * C]
        ff = jnp.concatenate([(nbr - xr).astype(jnp.bfloat16), xr_bf],
                             axis=1)
        acc = acc + lax.dot_general(w_bf, ff, (((1,), (1,)), ((), ())),
                                    preferred_element_type=jnp.float32)
    pre_ref[0] = acc / float(K)


def _conv(feat, xt, w, *, nblk=2):
    B, N, _ = feat.shape
    C = xt.shape[2]
    O = w.shape[0]
    bs = N // nblk
    return pl.pallas_call(
        _conv_body,
        grid=(B, nblk),
        in_specs=[
            pl.BlockSpec((1, bs, KPAD * C), lambda b, r: (b, r, 0)),
            pl.BlockSpec((1, bs, C), lambda b, r: (b, r, 0)),
            pl.BlockSpec((O, 2 * C), lambda b, r: (0, 0)),
        ],
        out_specs=pl.BlockSpec((1, O, bs), lambda b, r: (b, 0, r)),
        out_shape=jax.ShapeDtypeStruct((B, O, N), jnp.float32),
    )(feat, xt, w)


def _gcn_body(pre_ref, res_ref, out_ref):
    p = pre_ref[0]
    N = p.shape[1]
    mu = jnp.mean(p, axis=1, keepdims=True)
    dev = p - mu
    var = jnp.sum(dev * dev, axis=1, keepdims=True) / (N - 1)
    y = dev / jnp.sqrt(var + 0.001)
    y = jnp.maximum(y, 0.0)
    if res_ref is not None:
        y = y + res_ref[0]
    out_ref[0] = y


def _gcn(pre, residual):
    B, O, N = pre.shape
    if residual is None:
        def kern(pre_ref, out_ref):
            _gcn_body(pre_ref, None, out_ref)
        operands = (pre,)
        in_specs = [pl.BlockSpec((1, O, N), lambda b: (b, 0, 0))]
    else:
        kern = _gcn_body
        operands = (pre, residual)
        in_specs = [pl.BlockSpec((1, O, N), lambda b: (b, 0, 0)),
                    pl.BlockSpec((1, O, N), lambda b: (b, 0, 0))]
    return pl.pallas_call(
        kern,
        grid=(B,),
        in_specs=in_specs,
        out_specs=pl.BlockSpec((1, O, N), lambda b: (b, 0, 0)),
        out_shape=jax.ShapeDtypeStruct((B, O, N), jnp.float32),
    )(*operands)


def _layer(x, xt, w, residual):
    B, C, N = x.shape
    table = xt.reshape(B * N, C)
    idxs = [_topk(x[b:b + 1]) for b in range(B)]
    feats = [_sc_gather(table, idxs[b].reshape(N * KPAD) + b * N)
             for b in range(B)]
    outs = []
    for b in range(B):
        pre = _conv(feats[b].reshape(1, N, KPAD * C), xt[b:b + 1], w)
        outs.append(_gcn(pre, None if residual is None
                         else residual[b:b + 1]))
    return jnp.concatenate(outs, axis=0)


def kernel(x, W1, b1, W2, b2):
    del b1, b2
    xt = jnp.transpose(x, (0, 2, 1))
    h = _layer(x, xt, W1, None)
    ht = jnp.transpose(h, (0, 2, 1))
    return _layer(h, ht, W2, x)
